# Initial kernel scaffold; baseline (speedup 1.0000x reference)
import jax
import jax.numpy as jnp
from jax.experimental import pallas as pl


def kernel(x, adj, W_in, b_in, conv_W, conv_b, bn_gamma, bn_beta, bn_mean, bn_var, W_out, b_out):
    raise NotImplementedError("write your pallas kernel here")


# speedup vs baseline: 3.0759x; 3.0759x over previous
import functools

import jax
import jax.numpy as jnp
from jax.experimental import pallas as pl

_EPS = 1e-5


def _fwd(x_ref, adj_ref, w_in_ref, b_in_ref, conv_w_ref, conv_b_ref,
         g_ref, be_ref, mu_ref, var_ref, w_out_ref, b_out_ref,
         node_ref, graph_ref):
    f32 = jnp.float32
    a = adj_ref[0, 0]
    n = a.shape[0]
    ones = jnp.ones((n, 1), dtype=f32)
    cdim = (((0,), (0,)), ((), ()))

    deg = jax.lax.dot_general(a, ones, cdim, preferred_element_type=f32) + 1.0
    dinv = jax.lax.rsqrt(deg)
    s = jax.lax.dot_general(a, dinv, cdim, preferred_element_type=f32) + dinv
    u = dinv * s

    h0 = x_ref[0, 0] @ w_in_ref[...] + b_in_ref[...]
    g0 = h0 @ conv_w_ref[0]
    istd0 = jax.lax.rsqrt(var_ref[0:1] + _EPS)
    p = g0 * istd0 * g_ref[0:1]
    q = (conv_b_ref[0:1] - mu_ref[0:1]) * istd0 * g_ref[0:1] + be_ref[0:1]
    h1 = h0 + jnp.maximum(u * p + q, 0.0)

    v = dinv * (h1 @ conv_w_ref[1])
    agg = jax.lax.dot_general(a, v, cdim, preferred_element_type=f32) + v
    pre = dinv * agg + conv_b_ref[1:2]
    istd1 = jax.lax.rsqrt(var_ref[1:2] + _EPS)
    bn1 = (pre - mu_ref[1:2]) * istd1 * g_ref[1:2] + be_ref[1:2]
    h2 = h1 + jnp.maximum(bn1, 0.0)

    out = h2 @ w_out_ref[...] + b_out_ref[...]
    node_ref[0, 0] = out
    graph_ref[0, 0] = jnp.sum(out, axis=0, keepdims=True) * (1.0 / n)


def kernel(x, adj, W_in, b_in, conv_W, conv_b, bn_gamma, bn_beta, bn_mean,
           bn_var, W_out, b_out):
    B, T, Cin = x.shape
    N = adj.shape[-1]
    H = W_in.shape[1]
    Cout = W_out.shape[1]
    L = conv_W.shape[0]

    rep2 = lambda b, t: (0, 0)
    rep3 = lambda b, t: (0, 0, 0)
    node, graph = pl.pallas_call(
        _fwd,
        grid=(B, T),
        in_specs=[
            pl.BlockSpec((1, 1, 1, Cin), lambda b, t: (b, t, 0, 0)),
            pl.BlockSpec((1, 1, N, N), lambda b, t: (b, t, 0, 0)),
            pl.BlockSpec((Cin, H), rep2),
            pl.BlockSpec((1, H), rep2),
            pl.BlockSpec((L, H, H), rep3),
            pl.BlockSpec((L, H), rep2),
            pl.BlockSpec((L, H), rep2),
            pl.BlockSpec((L, H), rep2),
            pl.BlockSpec((L, H), rep2),
            pl.BlockSpec((L, H), rep2),
            pl.BlockSpec((H, Cout), rep2),
            pl.BlockSpec((1, Cout), rep2),
        ],
        out_specs=[
            pl.BlockSpec((1, 1, N, Cout), lambda b, t: (b, t, 0, 0)),
            pl.BlockSpec((1, 1, 1, Cout), lambda b, t: (b, t, 0, 0)),
        ],
        out_shape=[
            jax.ShapeDtypeStruct((B, T, N, Cout), jnp.float32),
            jax.ShapeDtypeStruct((B, T, 1, Cout), jnp.float32),
        ],
    )(x.reshape(B, T, 1, Cin), adj, W_in, b_in.reshape(1, H), conv_W, conv_b,
      bn_gamma, bn_beta, bn_mean, bn_var, W_out, b_out.reshape(1, Cout))
    return node, graph.reshape(B, T, Cout)


# speedup vs baseline: 3.0822x; 1.0020x over previous
import functools

import jax
import jax.numpy as jnp
from jax.experimental import pallas as pl

_EPS = 1e-5


def _fwd(x_ref, adj_ref, w_in_ref, b_in_ref, conv_w_ref, conv_b_ref,
         g_ref, be_ref, mu_ref, var_ref, w_out_ref, b_out_ref,
         node_ref, graph_ref):
    f32 = jnp.float32
    bf16 = jnp.bfloat16
    a = adj_ref[0, 0].astype(bf16)
    n = a.shape[0]
    ones = jnp.ones((n, 1), dtype=bf16)
    cdim = (((0,), (0,)), ((), ()))

    deg = jax.lax.dot_general(a, ones, cdim, preferred_element_type=f32) + 1.0
    dinv = jax.lax.rsqrt(deg)
    s = jax.lax.dot_general(a, dinv.astype(bf16), cdim,
                            preferred_element_type=f32) + dinv
    u = dinv * s

    h0 = x_ref[0, 0] @ w_in_ref[...] + b_in_ref[...]
    g0 = h0 @ conv_w_ref[0]
    istd0 = jax.lax.rsqrt(var_ref[0:1] + _EPS)
    p = g0 * istd0 * g_ref[0:1]
    q = (conv_b_ref[0:1] - mu_ref[0:1]) * istd0 * g_ref[0:1] + be_ref[0:1]
    h1 = h0 + jnp.maximum(u * p + q, 0.0)

    v = dinv * (h1 @ conv_w_ref[1])
    agg = jax.lax.dot_general(a, v.astype(bf16), cdim,
                              preferred_element_type=f32) + v
    pre = dinv * agg + conv_b_ref[1:2]
    istd1 = jax.lax.rsqrt(var_ref[1:2] + _EPS)
    bn1 = (pre - mu_ref[1:2]) * istd1 * g_ref[1:2] + be_ref[1:2]
    h2 = h1 + jnp.maximum(bn1, 0.0)

    out = h2 @ w_out_ref[...] + b_out_ref[...]
    node_ref[0, 0] = out
    graph_ref[0, 0] = jnp.sum(out, axis=0, keepdims=True) * (1.0 / n)


def kernel(x, adj, W_in, b_in, conv_W, conv_b, bn_gamma, bn_beta, bn_mean,
           bn_var, W_out, b_out):
    B, T, Cin = x.shape
    N = adj.shape[-1]
    H = W_in.shape[1]
    Cout = W_out.shape[1]
    L = conv_W.shape[0]

    rep2 = lambda b, t: (0, 0)
    rep3 = lambda b, t: (0, 0, 0)
    node, graph = pl.pallas_call(
        _fwd,
        grid=(B, T),
        in_specs=[
            pl.BlockSpec((1, 1, 1, Cin), lambda b, t: (b, t, 0, 0)),
            pl.BlockSpec((1, 1, N, N), lambda b, t: (b, t, 0, 0)),
            pl.BlockSpec((Cin, H), rep2),
            pl.BlockSpec((1, H), rep2),
            pl.BlockSpec((L, H, H), rep3),
            pl.BlockSpec((L, H), rep2),
            pl.BlockSpec((L, H), rep2),
            pl.BlockSpec((L, H), rep2),
            pl.BlockSpec((L, H), rep2),
            pl.BlockSpec((L, H), rep2),
            pl.BlockSpec((H, Cout), rep2),
            pl.BlockSpec((1, Cout), rep2),
        ],
        out_specs=[
            pl.BlockSpec((1, 1, N, Cout), lambda b, t: (b, t, 0, 0)),
            pl.BlockSpec((1, 1, 1, Cout), lambda b, t: (b, t, 0, 0)),
        ],
        out_shape=[
            jax.ShapeDtypeStruct((B, T, N, Cout), jnp.float32),
            jax.ShapeDtypeStruct((B, T, 1, Cout), jnp.float32),
        ],
    )(x.reshape(B, T, 1, Cin), adj, W_in, b_in.reshape(1, H), conv_W, conv_b,
      bn_gamma, bn_beta, bn_mean, bn_var, W_out, b_out.reshape(1, Cout))
    return node, graph.reshape(B, T, Cout)


# speedup vs baseline: 3.6267x; 1.1767x over previous
import functools

import jax
import jax.numpy as jnp
from jax.experimental import pallas as pl

_EPS = 1e-5


def _fwd(x_ref, adj_ref, w_in_ref, b_in_ref, conv_w_ref, conv_b_ref,
         g_ref, be_ref, mu_ref, var_ref, w_out_ref, b_out_ref,
         node_ref, graph_ref):
    f32 = jnp.float32
    bf16 = jnp.bfloat16
    a_f = adj_ref[0, 0]
    a = a_f.astype(bf16)
    n = a.shape[0]
    cdim = (((0,), (0,)), ((), ()))

    deg = jnp.sum(a_f, axis=0, keepdims=True) + 1.0
    dinv = jnp.transpose(jax.lax.rsqrt(deg), (1, 0))
    s = jax.lax.dot_general(a, dinv.astype(bf16), cdim,
                            preferred_element_type=f32) + dinv
    u = dinv * s

    h0 = x_ref[0, 0] @ w_in_ref[...] + b_in_ref[...]
    g0 = h0 @ conv_w_ref[0]
    istd0 = jax.lax.rsqrt(var_ref[0:1] + _EPS)
    p = g0 * istd0 * g_ref[0:1]
    q = (conv_b_ref[0:1] - mu_ref[0:1]) * istd0 * g_ref[0:1] + be_ref[0:1]
    h1 = h0 + jnp.maximum(u * p + q, 0.0)

    v = dinv * (h1 @ conv_w_ref[1])
    agg = jax.lax.dot_general(a, v.astype(bf16), cdim,
                              preferred_element_type=f32) + v
    pre = dinv * agg + conv_b_ref[1:2]
    istd1 = jax.lax.rsqrt(var_ref[1:2] + _EPS)
    bn1 = (pre - mu_ref[1:2]) * istd1 * g_ref[1:2] + be_ref[1:2]
    h2 = h1 + jnp.maximum(bn1, 0.0)

    out = h2 @ w_out_ref[...] + b_out_ref[...]
    node_ref[0, 0] = out
    graph_ref[0, 0] = jnp.sum(out, axis=0, keepdims=True) * (1.0 / n)


def kernel(x, adj, W_in, b_in, conv_W, conv_b, bn_gamma, bn_beta, bn_mean,
           bn_var, W_out, b_out):
    B, T, Cin = x.shape
    N = adj.shape[-1]
    H = W_in.shape[1]
    Cout = W_out.shape[1]
    L = conv_W.shape[0]

    rep2 = lambda b, t: (0, 0)
    rep3 = lambda b, t: (0, 0, 0)
    node, graph = pl.pallas_call(
        _fwd,
        grid=(B, T),
        in_specs=[
            pl.BlockSpec((1, 1, 1, Cin), lambda b, t: (b, t, 0, 0)),
            pl.BlockSpec((1, 1, N, N), lambda b, t: (b, t, 0, 0)),
            pl.BlockSpec((Cin, H), rep2),
            pl.BlockSpec((1, H), rep2),
            pl.BlockSpec((L, H, H), rep3),
            pl.BlockSpec((L, H), rep2),
            pl.BlockSpec((L, H), rep2),
            pl.BlockSpec((L, H), rep2),
            pl.BlockSpec((L, H), rep2),
            pl.BlockSpec((L, H), rep2),
            pl.BlockSpec((H, Cout), rep2),
            pl.BlockSpec((1, Cout), rep2),
        ],
        out_specs=[
            pl.BlockSpec((1, 1, N, Cout), lambda b, t: (b, t, 0, 0)),
            pl.BlockSpec((1, 1, 1, Cout), lambda b, t: (b, t, 0, 0)),
        ],
        out_shape=[
            jax.ShapeDtypeStruct((B, T, N, Cout), jnp.float32),
            jax.ShapeDtypeStruct((B, T, 1, Cout), jnp.float32),
        ],
    )(x.reshape(B, T, 1, Cin), adj, W_in, b_in.reshape(1, H), conv_W, conv_b,
      bn_gamma, bn_beta, bn_mean, bn_var, W_out, b_out.reshape(1, Cout))
    return node, graph.reshape(B, T, Cout)
